# baseline (speedup 1.0000x reference)
import jax
import jax.numpy as jnp
from jax import lax
from jax.experimental import pallas as pl
from jax.experimental.pallas import tpu as pltpu
from jax.experimental.pallas import tpu_sc as plsc

N = 10000
E = 320000
F = 128
SCALE = F ** (-0.5)
EPS = 1e-5

NC = 2
NS = 16
NW = NC * NS
RPT = 632
NP = NS * RPT
EPT = 10240
PAD = EPT - E // NW
C = 128
NCHUNK = EPT // C
PCH = 16
NPH = NCHUNK // PCH
PHE = PCH * C

_mesh = plsc.VectorSubcoreMesh(core_axis_name="c", subcore_axis_name="s")
_sc_params = pltpu.CompilerParams(needs_layout_passes=False)


def _proj_body(x_ref, w_ref, th_ref, ph_ref, wh_ref):
    x = x_ref[...]
    dn = (((1,), (1,)), ((), ()))
    th_ref[...] = lax.dot_general(x, w_ref[0:F, :], dn,
                                  preferred_element_type=jnp.float32)
    ph_ref[...] = lax.dot_general(x, w_ref[F:2 * F, :], dn,
                                  preferred_element_type=jnp.float32)
    wh_ref[...] = lax.dot_general(x, w_ref[2 * F:3 * F, :], dn,
                                  preferred_element_type=jnp.float32)


def _proj(x, w_cat):
    blk = 2000
    return pl.pallas_call(
        _proj_body,
        grid=(N // blk,),
        in_specs=[
            pl.BlockSpec((blk, F), lambda i: (i, 0)),
            pl.BlockSpec((3 * F, F), lambda i: (0, 0)),
        ],
        out_specs=[pl.BlockSpec((blk, F), lambda i: (i, 0))] * 3,
        out_shape=[jax.ShapeDtypeStruct((N, F), jnp.float32)] * 3,
    )(x, w_cat)


def _k2_body(th_hbm, ph_hbm, src_hbm, dst_hbm, ex_hbm, den_hbm,
             src_v, dst_v, th0, ph0, th1, ph1, ex_v, den_v,
             st0, sp0, st1, sp1):
    cid = lax.axis_index("c")
    sid = lax.axis_index("s")
    wid = sid * NC + cid
    pltpu.sync_copy(src_hbm.at[wid], src_v)
    pltpu.sync_copy(dst_hbm.at[wid], dst_v)

    zero16 = jnp.zeros((16,), jnp.float32)

    @pl.loop(0, NP, step=16)
    def _zero(i):
        den_v[pl.ds(i, 16)] = zero16

    lane = lax.iota(jnp.int32, 16)

    def fire(cc, th_b, ph_b, s_t, s_p):
        pltpu.async_copy(th_hbm.at[src_v.at[cc]], th_b, s_t)
        pltpu.async_copy(ph_hbm.at[dst_v.at[cc]], ph_b, s_p)

    def wait(th_b, ph_b, s_t, s_p):
        pltpu.make_async_copy(th_hbm.at[pl.ds(0, C)], th_b, s_t).wait()
        pltpu.make_async_copy(th_hbm.at[pl.ds(0, C)], ph_b, s_p).wait()

    def compute(cc, th_b, ph_b):
        @pl.loop(0, C, step=16)
        def _group(g):
            res = zero16
            for j in range(16):
                e = g + j
                acc = th_b[e, pl.ds(0, 16)] * ph_b[e, pl.ds(0, 16)]
                for k in range(1, 8):
                    acc = acc + (th_b[e, pl.ds(16 * k, 16)] *
                                 ph_b[e, pl.ds(16 * k, 16)])
                s = jnp.sum(acc)
                res = jnp.where(lane == j, s, res)
            ex16 = jnp.exp(res * SCALE)
            ex_v[pl.ds(cc * C + g, 16)] = ex16
            dst16 = dst_v[cc, pl.ds(g, 16)]
            plsc.addupdate_scatter(den_v, [dst16], ex16)

    fire(0, th0, ph0, st0, sp0)

    @pl.loop(0, NCHUNK, step=2)
    def _cc(cc):
        wait(th0, ph0, st0, sp0)
        fire(cc + 1, th1, ph1, st1, sp1)
        compute(cc, th0, ph0)
        wait(th1, ph1, st1, sp1)

        @pl.when(cc + 2 < NCHUNK)
        def _():
            fire(cc + 2, th0, ph0, st0, sp0)

        compute(cc + 1, th1, ph1)

    pltpu.sync_copy(ex_v, ex_hbm.at[wid])
    pltpu.sync_copy(den_v, den_hbm.at[wid])


def _k2(theta, phi, srcg, dstg):
    f = pl.kernel(
        _k2_body,
        out_type=[jax.ShapeDtypeStruct((NW, EPT), jnp.float32),
                  jax.ShapeDtypeStruct((NW, NP), jnp.float32)],
        mesh=_mesh,
        compiler_params=_sc_params,
        scratch_types=[
            pltpu.VMEM((NCHUNK, C), jnp.int32),
            pltpu.VMEM((NCHUNK, C), jnp.int32),
            pltpu.VMEM((C, F), jnp.float32),
            pltpu.VMEM((C, F), jnp.float32),
            pltpu.VMEM((C, F), jnp.float32),
            pltpu.VMEM((C, F), jnp.float32),
            pltpu.VMEM((EPT,), jnp.float32),
            pltpu.VMEM((NP,), jnp.float32),
            pltpu.SemaphoreType.DMA,
            pltpu.SemaphoreType.DMA,
            pltpu.SemaphoreType.DMA,
            pltpu.SemaphoreType.DMA,
        ],
    )
    return f(theta, phi, srcg, dstg)


def _den_body(p_ref, o_ref):
    o_ref[...] = jnp.sum(p_ref[...], axis=0, keepdims=True)


def _k3(den_p):
    return pl.pallas_call(
        _den_body,
        out_shape=jax.ShapeDtypeStruct((1, NP), jnp.float32),
    )(den_p)


def _k4_body(wh_hbm, src_hbm, dst_hbm, ex_hbm, den_hbm, z_hbm,
             src_v, dst_v, wh0, wh1, ex_v, den_v, z_sh, sg0, sg1, ss0, ss1):
    cid = lax.axis_index("c")
    sid = lax.axis_index("s")
    wid = sid * NC + cid
    pltpu.sync_copy(den_hbm.at[0], den_v)

    zero16 = jnp.zeros((16,), jnp.float32)

    @pl.loop(0, C)
    def _zrow(e):
        for k in range(8):
            wh0[e, pl.ds(16 * k, 16)] = zero16

    base = sid * RPT
    for r in range(RPT // C):
        pltpu.sync_copy(wh0, z_sh.at[pl.ds(base + r * C, C)])
    rem = RPT - (RPT // C) * C
    pltpu.sync_copy(wh0.at[pl.ds(0, rem)],
                    z_sh.at[pl.ds(base + (RPT // C) * C, rem)])
    plsc.subcore_barrier()

    def fire(cc, wh_b, sem):
        pltpu.async_copy(wh_hbm.at[src_v.at[cc]], wh_b, sem)

    def wait(wh_b, sem):
        pltpu.make_async_copy(wh_hbm.at[pl.ds(0, C)], wh_b, sem).wait()

    def fire_sc(cc, wh_b, sem):
        pltpu.async_copy(wh_b, z_sh.at[dst_v.at[cc]], sem, add=True)

    def wait_sc(wh_b, sem):
        pltpu.make_async_copy(wh_b, z_sh.at[pl.ds(0, C)], sem).wait()

    def scale(cc, wh_b):
        @pl.loop(0, C, step=16)
        def _group(g):
            dst16 = dst_v[cc, pl.ds(g, 16)]
            d16 = plsc.load_gather(den_v, [dst16])
            e16 = ex_v[pl.ds(cc * C + g, 16)]
            al16 = e16 / jnp.maximum(d16, 1e-38)
            for j in range(16):
                a = al16[j]
                for k in range(8):
                    wh_b[g + j, pl.ds(16 * k, 16)] = (
                        wh_b[g + j, pl.ds(16 * k, 16)] * a)

    @pl.loop(0, NPH)
    def _phase(p):
        pltpu.sync_copy(src_hbm.at[wid, p], src_v)
        pltpu.sync_copy(dst_hbm.at[wid, p], dst_v)
        pltpu.sync_copy(ex_hbm.at[wid, pl.ds(p * PHE, PHE)], ex_v)
        fire(0, wh0, sg0)
        fire(1, wh1, sg1)

        @pl.loop(0, PCH, step=2)
        def _cc(cc):
            wait(wh0, sg0)
            scale(cc, wh0)
            fire_sc(cc, wh0, ss0)
            wait(wh1, sg1)
            scale(cc + 1, wh1)
            fire_sc(cc + 1, wh1, ss1)

            @pl.when(cc + 2 < PCH)
            def _():
                wait_sc(wh0, ss0)
                fire(cc + 2, wh0, sg0)
                wait_sc(wh1, ss1)
                fire(cc + 3, wh1, sg1)

        wait_sc(wh0, ss0)
        wait_sc(wh1, ss1)

    plsc.subcore_barrier()
    pltpu.sync_copy(z_sh.at[pl.ds(base, RPT)], z_hbm.at[cid, pl.ds(base, RPT)])


def _k4(wh, srcg, dstg, ex, den):
    f = pl.kernel(
        _k4_body,
        out_type=jax.ShapeDtypeStruct((NC, NP, F), jnp.float32),
        mesh=_mesh,
        compiler_params=_sc_params,
        scratch_types=[
            pltpu.VMEM((PCH, C), jnp.int32),
            pltpu.VMEM((PCH, C), jnp.int32),
            pltpu.VMEM((C, F), jnp.float32),
            pltpu.VMEM((C, F), jnp.float32),
            pltpu.VMEM((PHE,), jnp.float32),
            pltpu.VMEM((NP,), jnp.float32),
            pltpu.VMEM_SHARED((NP, F), jnp.float32),
            pltpu.SemaphoreType.DMA,
            pltpu.SemaphoreType.DMA,
            pltpu.SemaphoreType.DMA,
            pltpu.SemaphoreType.DMA,
        ],
    )
    return f(wh, srcg, dstg, ex, den)


def _ln_body(z_ref, g_ref, b_ref, o_ref):
    zz = z_ref[0, 0:N, :] + z_ref[1, 0:N, :]
    mu = jnp.mean(zz, axis=1, keepdims=True)
    zc = zz - mu
    var = jnp.mean(zc * zc, axis=1, keepdims=True)
    o_ref[...] = zc * lax.rsqrt(var + EPS) * g_ref[...] + b_ref[...]


def _k5(z, gamma, beta):
    return pl.pallas_call(
        _ln_body,
        out_shape=jax.ShapeDtypeStruct((N, F), jnp.float32),
    )(z, gamma, beta)


def kernel(node_features, edge_index, W_fc, W_theta, W_phi, gamma, beta):
    w_cat = jnp.concatenate([W_theta, W_phi, W_fc], axis=0)
    theta, phi, wh = _proj(node_features, w_cat)
    src_r = edge_index[0].reshape(NW, E // NW)
    dst_r = edge_index[1].reshape(NW, E // NW)
    pad_dst = N + (jnp.arange(PAD, dtype=edge_index.dtype) % (NP - N))
    src_p = jnp.concatenate(
        [src_r, jnp.zeros((NW, PAD), edge_index.dtype)], axis=1)
    dst_p = jnp.concatenate(
        [dst_r, jnp.broadcast_to(pad_dst, (NW, PAD))], axis=1)
    srcg2 = src_p.reshape(NW, NCHUNK, C)
    dstg2 = dst_p.reshape(NW, NCHUNK, C)
    srcg4 = src_p.reshape(NW, NPH, PCH, C)
    dstg4 = dst_p.reshape(NW, NPH, PCH, C)
    ex, den_p = _k2(theta, phi, srcg2, dstg2)
    den = _k3(den_p)
    z = _k4(wh, srcg4, dstg4, ex, den)
    return _k5(z, gamma.reshape(1, F), beta.reshape(1, F))


# speedup vs baseline: 1.0772x; 1.0772x over previous
import jax
import jax.numpy as jnp
from jax import lax
from jax.experimental import pallas as pl
from jax.experimental.pallas import tpu as pltpu
from jax.experimental.pallas import tpu_sc as plsc

N = 10000
E = 320000
F = 128
SCALE = F ** (-0.5)
EPS = 1e-5

NC = 2
NS = 16
NW = NC * NS
RPT = 632
NP = NS * RPT
EPT = 10240
PAD = EPT - E // NW
C = 128
NCHUNK = EPT // C
PCH = 16
NPH = NCHUNK // PCH
PHE = PCH * C

_mesh = plsc.VectorSubcoreMesh(core_axis_name="c", subcore_axis_name="s")
_sc_params = pltpu.CompilerParams(needs_layout_passes=False)


def _proj_body(x_ref, w_ref, th_ref, ph_ref, wh_ref):
    x = x_ref[...]
    dn = (((1,), (1,)), ((), ()))
    th_ref[...] = lax.dot_general(x, w_ref[0:F, :], dn,
                                  preferred_element_type=jnp.float32)
    ph_ref[...] = lax.dot_general(x, w_ref[F:2 * F, :], dn,
                                  preferred_element_type=jnp.float32)
    wh_ref[...] = lax.dot_general(x, w_ref[2 * F:3 * F, :], dn,
                                  preferred_element_type=jnp.float32)


def _proj(x, w_cat):
    blk = 2000
    return pl.pallas_call(
        _proj_body,
        grid=(N // blk,),
        in_specs=[
            pl.BlockSpec((blk, F), lambda i: (i, 0)),
            pl.BlockSpec((3 * F, F), lambda i: (0, 0)),
        ],
        out_specs=[pl.BlockSpec((blk, F), lambda i: (i, 0))] * 3,
        out_shape=[jax.ShapeDtypeStruct((N, F), jnp.float32)] * 3,
    )(x, w_cat)


def _k2_body(th_hbm, ph_hbm, src_hbm, dst_hbm, ex_hbm, den_hbm,
             src_v, dst_v, th0, ph0, th1, ph1, th2, ph2, ex0, ex1, ex2, den_v,
             st0, sp0, st1, sp1, st2, sp2, se0, se1, se2):
    cid = lax.axis_index("c")
    sid = lax.axis_index("s")
    wid = sid * NC + cid
    pltpu.sync_copy(src_hbm.at[wid], src_v)
    pltpu.sync_copy(dst_hbm.at[wid], dst_v)

    zero16 = jnp.zeros((16,), jnp.float32)

    @pl.loop(0, NP, step=16)
    def _zero(i):
        den_v[pl.ds(i, 16)] = zero16

    lane = lax.iota(jnp.int32, 16)

    def fire(cc, th_b, ph_b, s_t, s_p):
        pltpu.async_copy(th_hbm.at[src_v.at[cc]], th_b, s_t)
        pltpu.async_copy(ph_hbm.at[dst_v.at[cc]], ph_b, s_p)

    def wait(th_b, ph_b, s_t, s_p):
        pltpu.make_async_copy(th_hbm.at[pl.ds(0, C)], th_b, s_t).wait()
        pltpu.make_async_copy(th_hbm.at[pl.ds(0, C)], ph_b, s_p).wait()

    def fire_ex(cc, ex_b, s_e):
        pltpu.async_copy(ex_b, ex_hbm.at[wid, pl.ds(cc * C, C)], s_e)

    def wait_ex(ex_b, s_e):
        pltpu.make_async_copy(ex_b, ex_hbm.at[wid, pl.ds(0, C)], s_e).wait()

    def compute(cc, th_b, ph_b, ex_b):
        @pl.loop(0, C, step=16)
        def _group(g):
            def edot(j, res):
                e = g + j
                acc = th_b[e, pl.ds(0, 16)] * ph_b[e, pl.ds(0, 16)]
                for k in range(1, 8):
                    acc = acc + (th_b[e, pl.ds(16 * k, 16)] *
                                 ph_b[e, pl.ds(16 * k, 16)])
                s = jnp.sum(acc)
                return jnp.where(lane == j, s, res)

            res = lax.fori_loop(0, 16, edot, zero16)
            ex16 = jnp.exp(res * SCALE)
            ex_b[pl.ds(g, 16)] = ex16
            dst16 = dst_v[cc, pl.ds(g, 16)]
            plsc.addupdate_scatter(den_v, [dst16], ex16)

    fire(0, th0, ph0, st0, sp0)
    fire(1, th1, ph1, st1, sp1)
    fire(2, th2, ph2, st2, sp2)

    @pl.loop(0, NCHUNK - 2, step=3)
    def _cc(cc):
        wait(th0, ph0, st0, sp0)

        @pl.when(cc >= 3)
        def _():
            wait_ex(ex0, se0)

        compute(cc, th0, ph0, ex0)
        fire_ex(cc, ex0, se0)
        fire(cc + 3, th0, ph0, st0, sp0)

        wait(th1, ph1, st1, sp1)

        @pl.when(cc >= 2)
        def _():
            wait_ex(ex1, se1)

        compute(cc + 1, th1, ph1, ex1)
        fire_ex(cc + 1, ex1, se1)
        fire(cc + 4, th1, ph1, st1, sp1)

        wait(th2, ph2, st2, sp2)

        @pl.when(cc >= 1)
        def _():
            wait_ex(ex2, se2)

        compute(cc + 2, th2, ph2, ex2)
        fire_ex(cc + 2, ex2, se2)

        @pl.when(cc + 5 < NCHUNK)
        def _():
            fire(cc + 5, th2, ph2, st2, sp2)

    wait(th0, ph0, st0, sp0)
    wait_ex(ex0, se0)
    compute(NCHUNK - 2, th0, ph0, ex0)
    fire_ex(NCHUNK - 2, ex0, se0)
    wait(th1, ph1, st1, sp1)
    wait_ex(ex1, se1)
    compute(NCHUNK - 1, th1, ph1, ex1)
    fire_ex(NCHUNK - 1, ex1, se1)

    wait_ex(ex0, se0)
    wait_ex(ex1, se1)
    wait_ex(ex2, se2)
    pltpu.sync_copy(den_v, den_hbm.at[wid])


def _k2(theta, phi, srcg, dstg):
    f = pl.kernel(
        _k2_body,
        out_type=[jax.ShapeDtypeStruct((NW, EPT), jnp.float32),
                  jax.ShapeDtypeStruct((NW, NP), jnp.float32)],
        mesh=_mesh,
        compiler_params=_sc_params,
        scratch_types=[
            pltpu.VMEM((NCHUNK, C), jnp.int32),
            pltpu.VMEM((NCHUNK, C), jnp.int32),
            pltpu.VMEM((C, F), jnp.float32),
            pltpu.VMEM((C, F), jnp.float32),
            pltpu.VMEM((C, F), jnp.float32),
            pltpu.VMEM((C, F), jnp.float32),
            pltpu.VMEM((C, F), jnp.float32),
            pltpu.VMEM((C, F), jnp.float32),
            pltpu.VMEM((C,), jnp.float32),
            pltpu.VMEM((C,), jnp.float32),
            pltpu.VMEM((C,), jnp.float32),
            pltpu.VMEM((NP,), jnp.float32),
            pltpu.SemaphoreType.DMA,
            pltpu.SemaphoreType.DMA,
            pltpu.SemaphoreType.DMA,
            pltpu.SemaphoreType.DMA,
            pltpu.SemaphoreType.DMA,
            pltpu.SemaphoreType.DMA,
            pltpu.SemaphoreType.DMA,
            pltpu.SemaphoreType.DMA,
            pltpu.SemaphoreType.DMA,
        ],
    )
    return f(theta, phi, srcg, dstg)


def _den_body(p_ref, o_ref):
    o_ref[...] = jnp.sum(p_ref[...], axis=0, keepdims=True)


def _k3(den_p):
    return pl.pallas_call(
        _den_body,
        out_shape=jax.ShapeDtypeStruct((1, NP), jnp.float32),
    )(den_p)


def _k4_body(wh_hbm, src_hbm, dst_hbm, ex_hbm, den_hbm, z_hbm,
             src_v, dst_v, wh0, wh1, ex_v, den_v, z_sh, sg0, sg1, ss0, ss1):
    cid = lax.axis_index("c")
    sid = lax.axis_index("s")
    wid = sid * NC + cid
    pltpu.sync_copy(den_hbm.at[0], den_v)

    zero16 = jnp.zeros((16,), jnp.float32)

    @pl.loop(0, C)
    def _zrow(e):
        for k in range(8):
            wh0[e, pl.ds(16 * k, 16)] = zero16

    base = sid * RPT
    for r in range(RPT // C):
        pltpu.sync_copy(wh0, z_sh.at[pl.ds(base + r * C, C)])
    rem = RPT - (RPT // C) * C
    pltpu.sync_copy(wh0.at[pl.ds(0, rem)],
                    z_sh.at[pl.ds(base + (RPT // C) * C, rem)])
    plsc.subcore_barrier()

    def fire(cc, wh_b, sem):
        pltpu.async_copy(wh_hbm.at[src_v.at[cc]], wh_b, sem)

    def wait(wh_b, sem):
        pltpu.make_async_copy(wh_hbm.at[pl.ds(0, C)], wh_b, sem).wait()

    def fire_sc(cc, wh_b, sem):
        pltpu.async_copy(wh_b, z_sh.at[dst_v.at[cc]], sem, add=True)

    def wait_sc(wh_b, sem):
        pltpu.make_async_copy(wh_b, z_sh.at[pl.ds(0, C)], sem).wait()

    def scale(cc, wh_b):
        @pl.loop(0, C, step=16)
        def _group(g):
            dst16 = dst_v[cc, pl.ds(g, 16)]
            d16 = plsc.load_gather(den_v, [dst16])
            e16 = ex_v[pl.ds(cc * C + g, 16)]
            al16 = e16 / jnp.maximum(d16, 1e-38)
            for j in range(16):
                a = al16[j]
                for k in range(8):
                    wh_b[g + j, pl.ds(16 * k, 16)] = (
                        wh_b[g + j, pl.ds(16 * k, 16)] * a)

    @pl.loop(0, NPH)
    def _phase(p):
        pltpu.sync_copy(src_hbm.at[wid, p], src_v)
        pltpu.sync_copy(dst_hbm.at[wid, p], dst_v)
        pltpu.sync_copy(ex_hbm.at[wid, pl.ds(p * PHE, PHE)], ex_v)
        fire(0, wh0, sg0)
        fire(1, wh1, sg1)

        @pl.loop(0, PCH, step=2)
        def _cc(cc):
            wait(wh0, sg0)
            scale(cc, wh0)
            fire_sc(cc, wh0, ss0)
            wait(wh1, sg1)
            scale(cc + 1, wh1)
            fire_sc(cc + 1, wh1, ss1)

            @pl.when(cc + 2 < PCH)
            def _():
                wait_sc(wh0, ss0)
                fire(cc + 2, wh0, sg0)
                wait_sc(wh1, ss1)
                fire(cc + 3, wh1, sg1)

        wait_sc(wh0, ss0)
        wait_sc(wh1, ss1)

    plsc.subcore_barrier()
    pltpu.sync_copy(z_sh.at[pl.ds(base, RPT)], z_hbm.at[cid, pl.ds(base, RPT)])


def _k4(wh, srcg, dstg, ex, den):
    f = pl.kernel(
        _k4_body,
        out_type=jax.ShapeDtypeStruct((NC, NP, F), jnp.float32),
        mesh=_mesh,
        compiler_params=_sc_params,
        scratch_types=[
            pltpu.VMEM((PCH, C), jnp.int32),
            pltpu.VMEM((PCH, C), jnp.int32),
            pltpu.VMEM((C, F), jnp.float32),
            pltpu.VMEM((C, F), jnp.float32),
            pltpu.VMEM((PHE,), jnp.float32),
            pltpu.VMEM((NP,), jnp.float32),
            pltpu.VMEM_SHARED((NP, F), jnp.float32),
            pltpu.SemaphoreType.DMA,
            pltpu.SemaphoreType.DMA,
            pltpu.SemaphoreType.DMA,
            pltpu.SemaphoreType.DMA,
        ],
    )
    return f(wh, srcg, dstg, ex, den)


def _ln_body(z_ref, g_ref, b_ref, o_ref):
    zz = z_ref[0, 0:N, :] + z_ref[1, 0:N, :]
    mu = jnp.mean(zz, axis=1, keepdims=True)
    zc = zz - mu
    var = jnp.mean(zc * zc, axis=1, keepdims=True)
    o_ref[...] = zc * lax.rsqrt(var + EPS) * g_ref[...] + b_ref[...]


def _k5(z, gamma, beta):
    return pl.pallas_call(
        _ln_body,
        out_shape=jax.ShapeDtypeStruct((N, F), jnp.float32),
    )(z, gamma, beta)


def kernel(node_features, edge_index, W_fc, W_theta, W_phi, gamma, beta):
    w_cat = jnp.concatenate([W_theta, W_phi, W_fc], axis=0)
    theta, phi, wh = _proj(node_features, w_cat)
    src_r = edge_index[0].reshape(NW, E // NW)
    dst_r = edge_index[1].reshape(NW, E // NW)
    pad_dst = N + (jnp.arange(PAD, dtype=edge_index.dtype) % (NP - N))
    src_p = jnp.concatenate(
        [src_r, jnp.zeros((NW, PAD), edge_index.dtype)], axis=1)
    dst_p = jnp.concatenate(
        [dst_r, jnp.broadcast_to(pad_dst, (NW, PAD))], axis=1)
    srcg2 = src_p.reshape(NW, NCHUNK, C)
    dstg2 = dst_p.reshape(NW, NCHUNK, C)
    srcg4 = src_p.reshape(NW, NPH, PCH, C)
    dstg4 = dst_p.reshape(NW, NPH, PCH, C)
    ex, den_p = _k2(theta, phi, srcg2, dstg2)
    den = _k3(den_p)
    z = _k4(wh, srcg4, dstg4, ex, den)
    return _k5(z, gamma.reshape(1, F), beta.reshape(1, F))
